# baseline (speedup 1.0000x reference)
import functools

import jax
import jax.numpy as jnp
from jax import lax
from jax.experimental import pallas as pl
from jax.experimental.pallas import tpu as pltpu
from jax.experimental.pallas import tpu_sc as plsc

N = 10000
IN_DIM = 128
NPAD = 10240
E = 320000
EPT = 10000
NTILES = 32
NB = 80
BE = 128
RPT = NPAD // NTILES
HID = 16
DUMMY = 10000

_MESH = plsc.VectorSubcoreMesh(core_axis_name="c", subcore_axis_name="s")


DW = 8


def _fill_tail(idx2d):
    iota = lax.iota(jnp.int32, 16)
    for j in range(7):
        idx2d[NB - 2, pl.ds(16 + 16 * j, 16)] = N + 16 * j + iota
    for j in range(8):
        idx2d[NB - 1, pl.ds(16 * j, 16)] = N + 112 + 16 * j + iota


def _stage_dst(dst_hbm, base, dst_v, sem):
    for k in range(NB - 2):
        pltpu.async_copy(dst_hbm.at[pl.ds(base + k * BE, BE)], dst_v.at[k],
                         sem)
    pltpu.async_copy(dst_hbm.at[pl.ds(base + EPT - 16, 16)],
                     dst_v.at[NB - 2, pl.ds(0, 16)], sem)
    for k in range(NB - 2):
        pltpu.make_async_copy(dst_hbm.at[pl.ds(0, BE)], dst_v.at[k],
                              sem).wait()
    pltpu.make_async_copy(dst_hbm.at[pl.ds(0, 16)],
                          dst_v.at[NB - 2, pl.ds(0, 16)], sem).wait()
    _fill_tail(dst_v)


def _deg_body(dst_hbm, zo_hbm, out0_hbm, out1_hbm, dst_v, zo_v, acc, sem):
    cid = lax.axis_index("c")
    sid = lax.axis_index("s")
    wid = cid * 16 + sid

    pltpu.sync_copy(zo_hbm, zo_v)
    pltpu.sync_copy(zo_v.at[pl.ds(BE, RPT)], acc.at[pl.ds(sid * RPT, RPT)])
    plsc.subcore_barrier()

    _stage_dst(dst_hbm, wid * EPT, dst_v, sem)

    def _scat(k, carry):
        pltpu.sync_copy(zo_v.at[pl.ds(0, BE)], acc.at[dst_v.at[k]], add=True)
        return carry

    lax.fori_loop(0, NB, _scat, 0)
    plsc.subcore_barrier()

    @pl.when(cid == 0)
    def _():
        pltpu.sync_copy(acc.at[pl.ds(sid * RPT, RPT)],
                        out0_hbm.at[pl.ds(sid * RPT, RPT)])

    @pl.when(cid == 1)
    def _():
        pltpu.sync_copy(acc.at[pl.ds(sid * RPT, RPT)],
                        out1_hbm.at[pl.ds(sid * RPT, RPT)])


_deg_call = functools.partial(
    pl.kernel,
    out_type=(jax.ShapeDtypeStruct((NPAD, DW), jnp.float32),
              jax.ShapeDtypeStruct((NPAD, DW), jnp.float32)),
    mesh=_MESH,
    compiler_params=pltpu.CompilerParams(use_tc_tiling_on_sc=False),
    scratch_types=[
        pltpu.VMEM((NB, BE), jnp.int32),
        pltpu.VMEM((BE + RPT, DW), jnp.float32),
        pltpu.VMEM_SHARED((NPAD, DW), jnp.float32),
        pltpu.SemaphoreType.DMA,
    ],
)(_deg_body)


NBUF = 5


def _prop_body(yt_hbm, src_hbm, dst_hbm, out0_hbm, out1_hbm, src1d,
               dst_v, rows_v, zbuf, acc, s0, s1, s2, s3, s4, sem):
    cid = lax.axis_index("c")
    sid = lax.axis_index("s")
    wid = cid * 16 + sid
    sems = (s0, s1, s2, s3, s4)

    def _fill_zero(i, carry):
        zbuf[i, :] = jnp.zeros((16,), jnp.float32)
        return carry

    lax.fori_loop(0, RPT, _fill_zero, 0)
    pltpu.sync_copy(zbuf, acc.at[pl.ds(sid * RPT, RPT)])
    plsc.subcore_barrier()

    pltpu.sync_copy(src_hbm.at[pl.ds(wid * EPT, EPT)], src1d.at[pl.ds(0, EPT)])
    iota = lax.iota(jnp.int32, 16)
    for j in range(15):
        src1d[pl.ds(EPT + 16 * j, 16)] = N + 16 * j + iota
    _stage_dst(dst_hbm, wid * EPT, dst_v, sem)

    for b in range(NBUF - 1):
        pltpu.async_copy(yt_hbm.at[src1d.at[pl.ds(b * BE, BE)]],
                         rows_v.at[b], sems[b])

    def _edge_group(g, carry):
        for b in range(NBUF):
            k = g * NBUF + b
            pltpu.make_async_copy(yt_hbm.at[src1d.at[pl.ds(0, BE)]],
                                  rows_v.at[b], sems[b]).wait()
            pltpu.sync_copy(rows_v.at[b], acc.at[dst_v.at[k]], add=True)
            nxt = k + NBUF - 1
            nb = (b + NBUF - 1) % NBUF

            @pl.when(nxt < NB)
            def _():
                pltpu.async_copy(yt_hbm.at[src1d.at[pl.ds(nxt * BE, BE)]],
                                 rows_v.at[nb], sems[nb])

        return carry

    lax.fori_loop(0, NB // NBUF, _edge_group, 0)
    plsc.subcore_barrier()

    @pl.when(cid == 0)
    def _():
        pltpu.sync_copy(acc.at[pl.ds(sid * RPT, RPT)],
                        out0_hbm.at[pl.ds(sid * RPT, RPT)])

    @pl.when(cid == 1)
    def _():
        pltpu.sync_copy(acc.at[pl.ds(sid * RPT, RPT)],
                        out1_hbm.at[pl.ds(sid * RPT, RPT)])


_prop_call = functools.partial(
    pl.kernel,
    out_type=(jax.ShapeDtypeStruct((NPAD, HID), jnp.float32),
              jax.ShapeDtypeStruct((NPAD, HID), jnp.float32)),
    mesh=_MESH,
    compiler_params=pltpu.CompilerParams(use_tc_tiling_on_sc=False),
    scratch_types=[
        pltpu.VMEM((NB * BE,), jnp.int32),
        pltpu.VMEM((NB, BE), jnp.int32),
        pltpu.VMEM((NBUF, BE, HID), jnp.float32),
        pltpu.VMEM((RPT, HID), jnp.float32),
        pltpu.VMEM_SHARED((NPAD, HID), jnp.float32),
        pltpu.SemaphoreType.DMA,
        pltpu.SemaphoreType.DMA,
        pltpu.SemaphoreType.DMA,
        pltpu.SemaphoreType.DMA,
        pltpu.SemaphoreType.DMA,
        pltpu.SemaphoreType.DMA,
    ],
)(_prop_body)


def _tc1_body(x_ref, w_ref, d0_ref, d1_ref, yt_ref, dinv_ref):
    deg = d0_ref[:, :1] + d1_ref[:, :1] + 1.0
    dinv = jnp.broadcast_to(lax.rsqrt(deg), (NPAD, HID))
    dinv_ref[...] = dinv
    xt = jnp.dot(x_ref[...], w_ref[...], preferred_element_type=jnp.float32)
    yt_ref[:N, :] = xt * dinv[:N, :]
    yt_ref[N:, :] = jnp.zeros((NPAD - N, HID), jnp.float32)


def _tc2_body(s0_ref, s1_ref, yt_ref, dinv_ref, w_ref, b_ref, out_ref):
    dinv = dinv_ref[...]
    h = jnp.maximum(dinv * (s0_ref[...] + s1_ref[...] + yt_ref[...])
                    + b_ref[...], 0.0)
    out_ref[...] = jnp.dot(h, w_ref[...],
                           preferred_element_type=jnp.float32) * dinv


def _tc3_body(s0_ref, s1_ref, yt_ref, dinv_ref, b_ref, out_ref):
    out_ref[...] = (dinv_ref[...] * (s0_ref[...] + s1_ref[...] + yt_ref[...])
                    + b_ref[...])


def _half_specs(minor):
    return [pl.BlockSpec((NPAD, minor), lambda i: (0, 0)),
            pl.BlockSpec((NPAD, minor), lambda i: (1, 0))]


def kernel(x, edge_index, W1, b1, W2, b2):
    src = edge_index[0]
    dst = edge_index[1]
    W2p = jnp.pad(W2, ((0, 0), (0, HID - W2.shape[1])))
    b1r = b1.reshape(1, HID)
    b2r = jnp.pad(b2, (0, HID - b2.shape[0])).reshape(1, HID)

    zo = jnp.concatenate([jnp.ones((BE, DW), jnp.float32),
                          jnp.zeros((RPT, DW), jnp.float32)])
    deg0, deg1 = _deg_call(dst, zo)

    yt1, dinv = pl.pallas_call(
        _tc1_body,
        out_shape=(jax.ShapeDtypeStruct((NPAD, HID), jnp.float32),
                   jax.ShapeDtypeStruct((NPAD, HID), jnp.float32)),
    )(x, W1, deg0, deg1)

    s1a, s1b = _prop_call(yt1, src, dst)

    rb = pl.BlockSpec((NPAD // 8, HID), lambda i: (i, 0))
    wb16 = pl.BlockSpec((HID, HID), lambda i: (0, 0))
    wb1 = pl.BlockSpec((1, HID), lambda i: (0, 0))
    yt2 = pl.pallas_call(
        _tc2_body,
        grid=(8,),
        in_specs=[rb, rb, rb, rb, wb16, wb1],
        out_specs=rb,
        out_shape=jax.ShapeDtypeStruct((NPAD, HID), jnp.float32),
    )(s1a, s1b, yt1, dinv, W2p, b1r)

    s2a, s2b = _prop_call(yt2, src, dst)

    out = pl.pallas_call(
        _tc3_body,
        grid=(8,),
        in_specs=[rb, rb, rb, rb, wb1],
        out_specs=rb,
        out_shape=jax.ShapeDtypeStruct((NPAD, HID), jnp.float32),
    )(s2a, s2b, yt2, dinv, b2r)

    return out[:N, :W2.shape[1]]


# speedup vs baseline: 1.0713x; 1.0713x over previous
import functools

import jax
import jax.numpy as jnp
from jax import lax
from jax.experimental import pallas as pl
from jax.experimental.pallas import tpu as pltpu
from jax.experimental.pallas import tpu_sc as plsc

N = 10000
IN_DIM = 128
NPAD = 10240
E = 320000
EPT = 10000
NTILES = 32
NB = 80
BE = 128
RPT = NPAD // NTILES
HID = 16
DUMMY = 10000

_MESH = plsc.VectorSubcoreMesh(core_axis_name="c", subcore_axis_name="s")


DW = 8


def _fill_tail(idx2d):
    iota = lax.iota(jnp.int32, 16)
    for j in range(7):
        idx2d[NB - 2, pl.ds(16 + 16 * j, 16)] = N + 16 * j + iota
    for j in range(8):
        idx2d[NB - 1, pl.ds(16 * j, 16)] = N + 112 + 16 * j + iota


def _stage_dst(ei_hbm, base, dst_v, sem):
    for k in range(NB - 2):
        pltpu.async_copy(ei_hbm.at[1, pl.ds(base + k * BE, BE)], dst_v.at[k],
                         sem)
    pltpu.async_copy(ei_hbm.at[1, pl.ds(base + EPT - 16, 16)],
                     dst_v.at[NB - 2, pl.ds(0, 16)], sem)
    for k in range(NB - 2):
        pltpu.make_async_copy(ei_hbm.at[1, pl.ds(0, BE)], dst_v.at[k],
                              sem).wait()
    pltpu.make_async_copy(ei_hbm.at[1, pl.ds(0, 16)],
                          dst_v.at[NB - 2, pl.ds(0, 16)], sem).wait()
    _fill_tail(dst_v)


def _deg_body(ei_hbm, zo_hbm, out0_hbm, out1_hbm, dst_v, zo_v, acc, sem):
    cid = lax.axis_index("c")
    sid = lax.axis_index("s")
    wid = cid * 16 + sid

    pltpu.sync_copy(zo_hbm, zo_v)
    pltpu.sync_copy(zo_v.at[pl.ds(BE, RPT)], acc.at[pl.ds(sid * RPT, RPT)])
    plsc.subcore_barrier()

    _stage_dst(ei_hbm, wid * EPT, dst_v, sem)

    def _scat(k, carry):
        pltpu.sync_copy(zo_v.at[pl.ds(0, BE)], acc.at[dst_v.at[k]], add=True)
        return carry

    lax.fori_loop(0, NB, _scat, 0)
    plsc.subcore_barrier()

    @pl.when(cid == 0)
    def _():
        pltpu.sync_copy(acc.at[pl.ds(sid * RPT, RPT)],
                        out0_hbm.at[pl.ds(sid * RPT, RPT)])

    @pl.when(cid == 1)
    def _():
        pltpu.sync_copy(acc.at[pl.ds(sid * RPT, RPT)],
                        out1_hbm.at[pl.ds(sid * RPT, RPT)])


_deg_call = functools.partial(
    pl.kernel,
    out_type=(jax.ShapeDtypeStruct((NPAD, DW), jnp.float32),
              jax.ShapeDtypeStruct((NPAD, DW), jnp.float32)),
    mesh=_MESH,
    compiler_params=pltpu.CompilerParams(use_tc_tiling_on_sc=False),
    scratch_types=[
        pltpu.VMEM((NB, BE), jnp.int32),
        pltpu.VMEM((BE + RPT, DW), jnp.float32),
        pltpu.VMEM_SHARED((NPAD, DW), jnp.float32),
        pltpu.SemaphoreType.DMA,
    ],
)(_deg_body)


NBUF = 5


def _prop_body(yt_hbm, ei_hbm, out0_hbm, out1_hbm, src1d,
               dst_v, rows_v, zbuf, acc, s0, s1, s2, s3, s4, sem):
    cid = lax.axis_index("c")
    sid = lax.axis_index("s")
    wid = cid * 16 + sid
    sems = (s0, s1, s2, s3, s4)

    def _fill_zero(i, carry):
        zbuf[i, :] = jnp.zeros((16,), jnp.float32)
        return carry

    lax.fori_loop(0, RPT, _fill_zero, 0)
    pltpu.sync_copy(zbuf, acc.at[pl.ds(sid * RPT, RPT)])
    plsc.subcore_barrier()

    pltpu.sync_copy(ei_hbm.at[0, pl.ds(wid * EPT, EPT)],
                    src1d.at[pl.ds(0, EPT)])
    iota = lax.iota(jnp.int32, 16)
    for j in range(15):
        src1d[pl.ds(EPT + 16 * j, 16)] = N + 16 * j + iota
    _stage_dst(ei_hbm, wid * EPT, dst_v, sem)

    for b in range(NBUF - 1):
        pltpu.async_copy(yt_hbm.at[src1d.at[pl.ds(b * BE, BE)]],
                         rows_v.at[b], sems[b])

    def _edge_group(g, carry):
        for b in range(NBUF):
            k = g * NBUF + b
            pltpu.make_async_copy(yt_hbm.at[src1d.at[pl.ds(0, BE)]],
                                  rows_v.at[b], sems[b]).wait()
            pltpu.sync_copy(rows_v.at[b], acc.at[dst_v.at[k]], add=True)
            nxt = k + NBUF - 1
            nb = (b + NBUF - 1) % NBUF

            @pl.when(nxt < NB)
            def _():
                pltpu.async_copy(yt_hbm.at[src1d.at[pl.ds(nxt * BE, BE)]],
                                 rows_v.at[nb], sems[nb])

        return carry

    lax.fori_loop(0, NB // NBUF, _edge_group, 0)
    plsc.subcore_barrier()

    @pl.when(cid == 0)
    def _():
        pltpu.sync_copy(acc.at[pl.ds(sid * RPT, RPT)],
                        out0_hbm.at[pl.ds(sid * RPT, RPT)])

    @pl.when(cid == 1)
    def _():
        pltpu.sync_copy(acc.at[pl.ds(sid * RPT, RPT)],
                        out1_hbm.at[pl.ds(sid * RPT, RPT)])


_prop_call = functools.partial(
    pl.kernel,
    out_type=(jax.ShapeDtypeStruct((NPAD, HID), jnp.float32),
              jax.ShapeDtypeStruct((NPAD, HID), jnp.float32)),
    mesh=_MESH,
    compiler_params=pltpu.CompilerParams(use_tc_tiling_on_sc=False),
    scratch_types=[
        pltpu.VMEM((NB * BE,), jnp.int32),
        pltpu.VMEM((NB, BE), jnp.int32),
        pltpu.VMEM((NBUF, BE, HID), jnp.float32),
        pltpu.VMEM((RPT, HID), jnp.float32),
        pltpu.VMEM_SHARED((NPAD, HID), jnp.float32),
        pltpu.SemaphoreType.DMA,
        pltpu.SemaphoreType.DMA,
        pltpu.SemaphoreType.DMA,
        pltpu.SemaphoreType.DMA,
        pltpu.SemaphoreType.DMA,
        pltpu.SemaphoreType.DMA,
    ],
)(_prop_body)


def _tc1_body(x_ref, w_ref, d0_ref, d1_ref, yt_ref, dinv_ref):
    deg = d0_ref[:, :1] + d1_ref[:, :1] + 1.0
    dinv = jnp.broadcast_to(lax.rsqrt(deg), (NPAD, HID))
    dinv_ref[...] = dinv
    xt = jnp.dot(x_ref[...], w_ref[...], preferred_element_type=jnp.float32)
    yt_ref[:N, :] = xt * dinv[:N, :]
    yt_ref[N:, :] = jnp.zeros((NPAD - N, HID), jnp.float32)


def _tc2_body(s0_ref, s1_ref, yt_ref, dinv_ref, w_ref, b_ref, out_ref):
    dinv = dinv_ref[...]
    h = jnp.maximum(dinv * (s0_ref[...] + s1_ref[...] + yt_ref[...])
                    + b_ref[...], 0.0)
    out_ref[...] = jnp.dot(h, w_ref[...],
                           preferred_element_type=jnp.float32) * dinv


def _tc3_body(s0_ref, s1_ref, yt_ref, dinv_ref, b_ref, out_ref):
    out_ref[...] = (dinv_ref[...] * (s0_ref[...] + s1_ref[...] + yt_ref[...])
                    + b_ref[...])


def _half_specs(minor):
    return [pl.BlockSpec((NPAD, minor), lambda i: (0, 0)),
            pl.BlockSpec((NPAD, minor), lambda i: (1, 0))]


def kernel(x, edge_index, W1, b1, W2, b2):
    W2p = jnp.pad(W2, ((0, 0), (0, HID - W2.shape[1])))
    b1r = b1.reshape(1, HID)
    b2r = jnp.pad(b2, (0, HID - b2.shape[0])).reshape(1, HID)

    zo = jnp.concatenate([jnp.ones((BE, DW), jnp.float32),
                          jnp.zeros((RPT, DW), jnp.float32)])
    deg0, deg1 = _deg_call(edge_index, zo)

    yt1, dinv = pl.pallas_call(
        _tc1_body,
        out_shape=(jax.ShapeDtypeStruct((NPAD, HID), jnp.float32),
                   jax.ShapeDtypeStruct((NPAD, HID), jnp.float32)),
    )(x, W1, deg0, deg1)

    s1a, s1b = _prop_call(yt1, edge_index)

    rb = pl.BlockSpec((NPAD // 8, HID), lambda i: (i, 0))
    wb16 = pl.BlockSpec((HID, HID), lambda i: (0, 0))
    wb1 = pl.BlockSpec((1, HID), lambda i: (0, 0))
    yt2 = pl.pallas_call(
        _tc2_body,
        grid=(8,),
        in_specs=[rb, rb, rb, rb, wb16, wb1],
        out_specs=rb,
        out_shape=jax.ShapeDtypeStruct((NPAD, HID), jnp.float32),
    )(s1a, s1b, yt1, dinv, W2p, b1r)

    s2a, s2b = _prop_call(yt2, edge_index)

    out = pl.pallas_call(
        _tc3_body,
        grid=(8,),
        in_specs=[rb, rb, rb, rb, wb1],
        out_specs=rb,
        out_shape=jax.ShapeDtypeStruct((NPAD, HID), jnp.float32),
    )(s2a, s2b, yt2, dinv, b2r)

    return out[:N, :W2.shape[1]]


# speedup vs baseline: 1.4476x; 1.3512x over previous
import functools

import numpy as np

import jax
import jax.numpy as jnp
from jax import lax
from jax.experimental import pallas as pl
from jax.experimental.pallas import tpu as pltpu
from jax.experimental.pallas import tpu_sc as plsc

N = 10000
IN_DIM = 128
NPAD = 10240
E = 320000
EPT = 10000
SELF = 320
NTILES = 32
NB = 81
BE = 128
RPT = NPAD // NTILES
HID = 16

_MESH = plsc.VectorSubcoreMesh(core_axis_name="c", subcore_axis_name="s")

_S_np = np.zeros((2, 128, 128), np.float32)
for _h in range(2):
    for _q in range(8):
        _S_np[_h, (_h * 8 + _q) * 8, _q * 16:(_q + 1) * 16] = 1.0
PACK = NPAD * HID // 128


DW = 8


def _fill_tail(idx2d, sbase):
    iota = lax.iota(jnp.int32, 16)
    for j in range(7):
        idx2d[78, pl.ds(16 + 16 * j, 16)] = sbase + 16 * j + iota
    for j in range(8):
        idx2d[79, pl.ds(16 * j, 16)] = sbase + 112 + 16 * j + iota
    for j in range(5):
        idx2d[80, pl.ds(16 * j, 16)] = sbase + 240 + 16 * j + iota
    for j in range(3):
        idx2d[80, pl.ds(80 + 16 * j, 16)] = N + 16 * j + iota


def _stage_dst(ei_hbm, base, sbase, dst_v, sem):
    for k in range(78):
        pltpu.async_copy(ei_hbm.at[1, pl.ds(base + k * BE, BE)], dst_v.at[k],
                         sem)
    pltpu.async_copy(ei_hbm.at[1, pl.ds(base + EPT - 16, 16)],
                     dst_v.at[78, pl.ds(0, 16)], sem)
    for k in range(78):
        pltpu.make_async_copy(ei_hbm.at[1, pl.ds(0, BE)], dst_v.at[k],
                              sem).wait()
    pltpu.make_async_copy(ei_hbm.at[1, pl.ds(0, 16)],
                          dst_v.at[78, pl.ds(0, 16)], sem).wait()
    _fill_tail(dst_v, sbase)


def _deg_body(ei_hbm, zo_hbm, out0_hbm, out1_hbm, dst_v, zo_v, acc, sem):
    cid = lax.axis_index("c")
    sid = lax.axis_index("s")
    wid = cid * 16 + sid

    pltpu.sync_copy(zo_hbm, zo_v)
    pltpu.sync_copy(zo_v.at[pl.ds(BE, RPT)], acc.at[pl.ds(sid * RPT, RPT)])
    plsc.subcore_barrier()

    _stage_dst(ei_hbm, wid * EPT, wid * SELF, dst_v, sem)

    def _scat(k, carry):
        pltpu.sync_copy(zo_v.at[pl.ds(0, BE)], acc.at[dst_v.at[k]], add=True)
        return carry

    lax.fori_loop(0, NB, _scat, 0)
    plsc.subcore_barrier()

    @pl.when(cid == 0)
    def _():
        pltpu.sync_copy(acc.at[pl.ds(sid * RPT, RPT)],
                        out0_hbm.at[pl.ds(sid * RPT, RPT)])

    @pl.when(cid == 1)
    def _():
        pltpu.sync_copy(acc.at[pl.ds(sid * RPT, RPT)],
                        out1_hbm.at[pl.ds(sid * RPT, RPT)])


_deg_call = functools.partial(
    pl.kernel,
    out_type=(jax.ShapeDtypeStruct((NPAD, DW), jnp.float32),
              jax.ShapeDtypeStruct((NPAD, DW), jnp.float32)),
    mesh=_MESH,
    compiler_params=pltpu.CompilerParams(use_tc_tiling_on_sc=False),
    scratch_types=[
        pltpu.VMEM((NB, BE), jnp.int32),
        pltpu.VMEM((BE + RPT, DW), jnp.float32),
        pltpu.VMEM_SHARED((NPAD, DW), jnp.float32),
        pltpu.SemaphoreType.DMA,
    ],
)(_deg_body)


NBUF = 16


def _prop_body(yt_hbm, ei_hbm, out0_hbm, out1_hbm, src1d,
               dst_v, rows_v, zbuf, acc, *sems_and_sem):
    cid = lax.axis_index("c")
    sid = lax.axis_index("s")
    wid = cid * 16 + sid
    sems = sems_and_sem[:NBUF]
    sem = sems_and_sem[NBUF]

    def _fill_zero(i, carry):
        zbuf[i, :] = jnp.zeros((16,), jnp.float32)
        return carry

    lax.fori_loop(0, RPT, _fill_zero, 0)
    pltpu.sync_copy(zbuf, acc.at[pl.ds(sid * RPT, RPT)])
    plsc.subcore_barrier()

    pltpu.sync_copy(ei_hbm.at[0, pl.ds(wid * EPT, EPT)],
                    src1d.at[pl.ds(0, EPT)])
    iota = lax.iota(jnp.int32, 16)
    for j in range(20):
        src1d[pl.ds(EPT + 16 * j, 16)] = wid * SELF + 16 * j + iota
    for j in range(3):
        src1d[pl.ds(EPT + SELF + 16 * j, 16)] = N + 16 * j + iota
    _stage_dst(ei_hbm, wid * EPT, wid * SELF, dst_v, sem)

    for b in range(NBUF - 1):
        pltpu.async_copy(yt_hbm.at[src1d.at[pl.ds(b * BE, BE)]],
                         rows_v.at[b], sems[b])

    def _edge_group(g, carry):
        for b in range(NBUF):
            k = g * NBUF + b
            pltpu.make_async_copy(yt_hbm.at[src1d.at[pl.ds(0, BE)]],
                                  rows_v.at[b], sems[b]).wait()
            pltpu.sync_copy(rows_v.at[b], acc.at[dst_v.at[k]], add=True)
            nxt = k + NBUF - 1
            nb = (b + NBUF - 1) % NBUF

            @pl.when(nxt < NB)
            def _():
                pltpu.async_copy(yt_hbm.at[src1d.at[pl.ds(nxt * BE, BE)]],
                                 rows_v.at[nb], sems[nb])

        return carry

    lax.fori_loop(0, (NB - 1) // NBUF, _edge_group, 0)
    lb = (NB - 1) % NBUF
    pltpu.make_async_copy(yt_hbm.at[src1d.at[pl.ds(0, BE)]],
                          rows_v.at[lb], sems[lb]).wait()
    pltpu.sync_copy(rows_v.at[lb], acc.at[dst_v.at[NB - 1]], add=True)
    plsc.subcore_barrier()

    @pl.when(cid == 0)
    def _():
        pltpu.sync_copy(acc.at[pl.ds(sid * RPT, RPT)],
                        out0_hbm.at[pl.ds(sid * RPT, RPT)])

    @pl.when(cid == 1)
    def _():
        pltpu.sync_copy(acc.at[pl.ds(sid * RPT, RPT)],
                        out1_hbm.at[pl.ds(sid * RPT, RPT)])


_prop_call = functools.partial(
    pl.kernel,
    out_type=(jax.ShapeDtypeStruct((NPAD, HID), jnp.float32),
              jax.ShapeDtypeStruct((NPAD, HID), jnp.float32)),
    mesh=_MESH,
    compiler_params=pltpu.CompilerParams(use_tc_tiling_on_sc=False),
    scratch_types=[
        pltpu.VMEM((NB * BE,), jnp.int32),
        pltpu.VMEM((NB, BE), jnp.int32),
        pltpu.VMEM((NBUF, BE, HID), jnp.float32),
        pltpu.VMEM((RPT, HID), jnp.float32),
        pltpu.VMEM_SHARED((NPAD, HID), jnp.float32),
        pltpu.SemaphoreType.DMA,
        pltpu.SemaphoreType.DMA,
        pltpu.SemaphoreType.DMA,
        pltpu.SemaphoreType.DMA,
        pltpu.SemaphoreType.DMA,
        pltpu.SemaphoreType.DMA,
        pltpu.SemaphoreType.DMA,
        pltpu.SemaphoreType.DMA,
        pltpu.SemaphoreType.DMA,
        pltpu.SemaphoreType.DMA,
        pltpu.SemaphoreType.DMA,
        pltpu.SemaphoreType.DMA,
        pltpu.SemaphoreType.DMA,
        pltpu.SemaphoreType.DMA,
        pltpu.SemaphoreType.DMA,
        pltpu.SemaphoreType.DMA,
        pltpu.SemaphoreType.DMA,
    ],
)(_prop_body)


def _tc1_body(x_ref, w_ref, d0_ref, d1_ref, dp0_ref, dp1_ref, s0_ref,
              s1_ref, yt_ref, dinvp_ref):
    deg = d0_ref[:, :1] + d1_ref[:, :1]
    dinv = jnp.broadcast_to(lax.rsqrt(deg), (NPAD, HID))
    xt = jnp.dot(x_ref[...], w_ref[...], preferred_element_type=jnp.float32)
    yt_ref[:N, :] = xt * dinv[:N, :]
    yt_ref[N:, :] = jnp.zeros((NPAD - N, HID), jnp.float32)
    rp = lax.rsqrt(dp0_ref[...] + dp1_ref[...])
    dinvp = jnp.stack(
        [jnp.dot(rp, s0_ref[...], preferred_element_type=jnp.float32),
         jnp.dot(rp, s1_ref[...], preferred_element_type=jnp.float32)],
        axis=1).reshape(PACK, 128)
    dinvp_ref[...] = dinvp


def _tc2_body(s0_ref, s1_ref, dinvp_ref, w8_ref, b_ref, out_ref):
    dinvp = dinvp_ref[...]
    h = jnp.maximum(dinvp * (s0_ref[...] + s1_ref[...]) + b_ref[...], 0.0)
    out_ref[...] = jnp.dot(h, w8_ref[...],
                           preferred_element_type=jnp.float32) * dinvp


def _tc3_body(s0_ref, s1_ref, dinvp_ref, b_ref, out_ref):
    out_ref[...] = dinvp_ref[...] * (s0_ref[...] + s1_ref[...]) + b_ref[...]


def kernel(x, edge_index, W1, b1, W2, b2):
    W2p = jnp.pad(W2, ((0, 0), (0, HID - W2.shape[1])))
    W8 = jnp.kron(jnp.eye(8, dtype=jnp.float32), W2p)
    b1t = jnp.tile(b1, 8).reshape(1, 128)
    b2t = jnp.tile(jnp.pad(b2, (0, HID - b2.shape[0])), 8).reshape(1, 128)

    zo = jnp.concatenate([jnp.ones((BE, DW), jnp.float32),
                          jnp.zeros((RPT, DW), jnp.float32)])
    deg0, deg1 = _deg_call(edge_index, zo)

    dp0 = deg0.reshape(NPAD * DW // 128, 128)
    dp1 = deg1.reshape(NPAD * DW // 128, 128)
    yt1, dinvp = pl.pallas_call(
        _tc1_body,
        out_shape=(jax.ShapeDtypeStruct((NPAD, HID), jnp.float32),
                   jax.ShapeDtypeStruct((PACK, 128), jnp.float32)),
    )(x, W1, deg0, deg1, dp0, dp1,
      jnp.asarray(_S_np[0]), jnp.asarray(_S_np[1]))

    s1a, s1b = _prop_call(yt1, edge_index)

    rb = pl.BlockSpec((PACK // 8, 128), lambda i: (i, 0))
    wb128 = pl.BlockSpec((128, 128), lambda i: (0, 0))
    wb1 = pl.BlockSpec((1, 128), lambda i: (0, 0))
    yt2p = pl.pallas_call(
        _tc2_body,
        grid=(8,),
        in_specs=[rb, rb, rb, wb128, wb1],
        out_specs=rb,
        out_shape=jax.ShapeDtypeStruct((PACK, 128), jnp.float32),
    )(s1a.reshape(PACK, 128), s1b.reshape(PACK, 128), dinvp, W8, b1t)

    s2a, s2b = _prop_call(yt2p.reshape(NPAD, HID), edge_index)

    out = pl.pallas_call(
        _tc3_body,
        grid=(8,),
        in_specs=[rb, rb, rb, wb1],
        out_specs=rb,
        out_shape=jax.ShapeDtypeStruct((PACK, 128), jnp.float32),
    )(s2a.reshape(PACK, 128), s2b.reshape(PACK, 128), dinvp, b2t)

    return out[:N * HID // 128].reshape(N, HID)[:, :W2.shape[1]]
